# Initial kernel scaffold; baseline (speedup 1.0000x reference)
import jax
import jax.numpy as jnp
from jax.experimental import pallas as pl


def kernel(feat_user, feat_item, ui_src, ui_dst, norm_ui, norm_iu, norm_user, norm_item, W1_w, W1_b, W2_w, W2_b):
    raise NotImplementedError("write your pallas kernel here")


# speedup vs baseline: 1.5519x; 1.5519x over previous
import functools

import jax
import jax.numpy as jnp
from jax import lax
from jax.experimental import pallas as pl
from jax.experimental.pallas import tpu as pltpu
from jax.experimental.pallas import tpu_sc as plsc

N = 5000
E = 320000
D = 128
W = 2 * D
RPT = 320
NPAD = 16 * RPT
ACC_ROWS = RPT + 8
SCAN = 1280
NW = E // SCAN
CAP = SCAN + 16


def _sc_edge_kernel(tbl_hbm, gidx_hbm, sidx_hbm, alpha_hbm, zeros_hbm,
                    out_hbm, acc_v, sin_v, gin_v, ain_v, cg_v, ca_v, cl_v,
                    rows_v):
    c = lax.axis_index("c")
    s = lax.axis_index("s")
    lo = s * RPT
    lanes = lax.iota(jnp.int32, 16)

    pltpu.sync_copy(zeros_hbm, acc_v)

    def accum_batch(bo):
        pltpu.sync_copy(tbl_hbm.at[cg_v.at[pl.ds(bo, 16)]], rows_v)
        av = ca_v[pl.ds(bo, 16)]
        lsv = cl_v[pl.ds(bo, 16)]
        for l in range(16):
            a = av[l]
            ls = lsv[l]
            for j in range(16):
                x = rows_v[l, pl.ds(j * 16, 16)]
                if j < 8:
                    x = x * a
                plsc.addupdate(acc_v.at[ls, pl.ds(j * 16, 16)], x)

    def win_body(w, cnt):
        off = pl.multiple_of(c * E + w * SCAN, SCAN)
        pltpu.sync_copy(sidx_hbm.at[pl.ds(off, SCAN)], sin_v)
        pltpu.sync_copy(gidx_hbm.at[pl.ds(off, SCAN)], gin_v)
        pltpu.sync_copy(alpha_hbm.at[pl.ds(off, SCAN)], ain_v)

        def grp(g, cnt):
            sl = pl.ds(g * 16, 16)
            sv = sin_v[sl]
            m = (sv >= lo) & (sv < lo + RPT)
            pos = cnt + plsc.cumsum(jnp.where(m, 1, 0)) - 1
            plsc.store_scatter(cg_v, [pos], gin_v[sl], mask=m)
            plsc.store_scatter(ca_v, [pos], ain_v[sl], mask=m)
            plsc.store_scatter(cl_v, [pos], sv - lo, mask=m)
            return cnt + plsc.all_reduce_population_count(m)[0]

        cnt = lax.fori_loop(0, SCAN // 16, grp, cnt, unroll=False)

        nb = cnt // 16

        def batch(b, carry):
            accum_batch(b * 16)
            return carry

        lax.fori_loop(0, nb, batch, 0, unroll=False)

        rem = cnt - nb * 16
        mv = lanes < rem
        src = pl.ds(nb * 16, 16)
        cg_v[pl.ds(0, 16)] = jnp.where(mv, cg_v[src], 0)
        ca_v[pl.ds(0, 16)] = jnp.where(mv, ca_v[src], 0.0)
        cl_v[pl.ds(0, 16)] = jnp.where(mv, cl_v[src], RPT)
        return rem

    cnt = lax.fori_loop(0, NW, win_body, 0, unroll=False)

    @pl.when(cnt > 0)
    def _():
        accum_batch(0)

    pltpu.sync_copy(acc_v.at[pl.ds(0, RPT)], out_hbm.at[c, pl.ds(lo, RPT)])


@jax.jit
def _sc_edge_call(tbl, gidx, sidx, alpha, zeros):
    mesh = plsc.VectorSubcoreMesh(core_axis_name="c", subcore_axis_name="s")
    kern = functools.partial(
        pl.kernel,
        mesh=mesh,
        compiler_params=pltpu.CompilerParams(needs_layout_passes=False),
        out_type=jax.ShapeDtypeStruct((2, NPAD, W), jnp.float32),
        scratch_types=[
            pltpu.VMEM((ACC_ROWS, W), jnp.float32),
            pltpu.VMEM((SCAN,), jnp.int32),
            pltpu.VMEM((SCAN,), jnp.int32),
            pltpu.VMEM((SCAN,), jnp.float32),
            pltpu.VMEM((CAP,), jnp.int32),
            pltpu.VMEM((CAP,), jnp.float32),
            pltpu.VMEM((CAP,), jnp.int32),
            pltpu.VMEM((16, W), jnp.float32),
        ],
    )(_sc_edge_kernel)
    return kern(tbl, gidx, sidx, alpha, zeros)


def _tc_finish_kernel(feat_ref, ns_ref, accA_ref, accM_ref,
                      w1t_ref, w2t_ref, b_ref, out_ref):
    fA = feat_ref[...] + accA_ref[...]
    fM = ns_ref[...] * accM_ref[...]
    h = jnp.dot(fA, w1t_ref[...], preferred_element_type=jnp.float32)
    h = h + jnp.dot(fM, w2t_ref[...], preferred_element_type=jnp.float32)
    h = h + b_ref[...]
    h = jnp.where(h >= 0, h, 0.2 * h)
    nrm = jnp.maximum(jnp.sqrt(jnp.sum(h * h, axis=1, keepdims=True)), 1e-12)
    out_ref[...] = h / nrm


@jax.jit
def _tc_finish_call(feat, ns, accA, accM, w1t, w2t, b):
    R = 1000
    nblk = (2 * N) // R
    blk = lambda i: (i, 0)
    return pl.pallas_call(
        _tc_finish_kernel,
        grid=(nblk,),
        in_specs=[
            pl.BlockSpec((R, D), blk),
            pl.BlockSpec((R, D), blk),
            pl.BlockSpec((R, D), blk),
            pl.BlockSpec((R, D), blk),
            pl.BlockSpec((D, D), lambda i: (0, 0)),
            pl.BlockSpec((D, D), lambda i: (0, 0)),
            pl.BlockSpec((1, D), lambda i: (0, 0)),
        ],
        out_specs=pl.BlockSpec((R, D), blk),
        out_shape=jax.ShapeDtypeStruct((2 * N, D), jnp.float32),
    )(feat, ns, accA, accM, w1t, w2t, b)


def kernel(feat_user, feat_item, ui_src, ui_dst, norm_ui, norm_iu,
           norm_user, norm_item, W1_w, W1_b, W2_w, W2_b):
    fu_s = feat_user * norm_user
    fi_s = feat_item * norm_item
    tbl = jnp.concatenate([
        jnp.concatenate([feat_item, fi_s], axis=1),
        jnp.concatenate([feat_user, fu_s], axis=1),
    ], axis=0)
    gidx = jnp.concatenate([ui_dst, ui_src + N])
    sidx = jnp.concatenate([ui_src, ui_dst])
    alpha = jnp.concatenate([norm_iu[:, 0], norm_ui[:, 0]])
    zeros = jnp.zeros((ACC_ROWS, W), jnp.float32)

    acc = _sc_edge_call(tbl, gidx, sidx, alpha, zeros)
    accA = acc[:, :N, :D].reshape(2 * N, D)
    accM = acc[:, :N, D:].reshape(2 * N, D)

    feat = jnp.concatenate([feat_user, feat_item], axis=0)
    ns = jnp.concatenate([fu_s, fi_s], axis=0)
    h = _tc_finish_call(feat, ns, accA, accM, W1_w.T, W2_w.T,
                        (W1_b + W2_b).reshape(1, D))
    return h[:N], h[N:]


# speedup vs baseline: 1.5821x; 1.0194x over previous
import functools

import jax
import jax.numpy as jnp
from jax import lax
from jax.experimental import pallas as pl
from jax.experimental.pallas import tpu as pltpu
from jax.experimental.pallas import tpu_sc as plsc

N = 5000
E = 320000
D = 128
W = 2 * D
RPT = 320
NPAD = 16 * RPT
ACC_ROWS = RPT + 8
SCAN = 3200
NW = E // SCAN
CAP = SCAN + 16


def _sc_edge_kernel(tbl_hbm, gidx_hbm, sidx_hbm, alpha_hbm, zeros_hbm,
                    out_hbm, acc_v, sin_a, gin_a, ain_a, sin_b, gin_b, ain_b,
                    cg_v, ca_v, cl_v, rows_a, rows_b,
                    sem_wa, sem_wb, sem_ga, sem_gb):
    c = lax.axis_index("c")
    s = lax.axis_index("s")
    lo = s * RPT
    lanes = lax.iota(jnp.int32, 16)

    pltpu.sync_copy(zeros_hbm, acc_v)

    def win_off(w):
        return pl.multiple_of(c * E + w * SCAN, SCAN)

    def start_win(w, bufs, sem):
        sv, gv, av = bufs
        off = win_off(w)
        pltpu.async_copy(sidx_hbm.at[pl.ds(off, SCAN)], sv, sem)
        pltpu.async_copy(gidx_hbm.at[pl.ds(off, SCAN)], gv, sem)
        pltpu.async_copy(alpha_hbm.at[pl.ds(off, SCAN)], av, sem)

    def wait_win(w, bufs, sem):
        sv, gv, av = bufs
        off = win_off(w)
        pltpu.make_async_copy(sidx_hbm.at[pl.ds(off, SCAN)], sv, sem).wait()
        pltpu.make_async_copy(gidx_hbm.at[pl.ds(off, SCAN)], gv, sem).wait()
        pltpu.make_async_copy(alpha_hbm.at[pl.ds(off, SCAN)], av, sem).wait()

    def start_gather(bo, rows, sem):
        pltpu.async_copy(tbl_hbm.at[cg_v.at[pl.ds(bo, 16)]], rows, sem)

    def wait_gather(bo, rows, sem):
        pltpu.make_async_copy(tbl_hbm.at[cg_v.at[pl.ds(bo, 16)]], rows,
                              sem).wait()

    def accum_rows(bo, rows):
        av = ca_v[pl.ds(bo, 16)]
        lsv = cl_v[pl.ds(bo, 16)]
        for l in range(16):
            a = av[l]
            ls = lsv[l]
            for j in range(16):
                x = rows[l, pl.ds(j * 16, 16)]
                if j < 8:
                    x = x * a
                plsc.addupdate(acc_v.at[ls, pl.ds(j * 16, 16)], x)

    def do_window(w, cnt, cur_bufs, cur_sem, nxt_w, nxt_bufs, nxt_sem):
        sin_v, gin_v, ain_v = cur_bufs
        wait_win(w, cur_bufs, cur_sem)

        @pl.when(nxt_w < NW)
        def _():
            start_win(nxt_w, nxt_bufs, nxt_sem)

        def grp(g, cnt):
            sl = pl.ds(g * 16, 16)
            sv = sin_v[sl]
            m = (sv >= lo) & (sv < lo + RPT)
            pos = cnt + plsc.cumsum(jnp.where(m, 1, 0)) - 1
            plsc.store_scatter(cg_v, [pos], gin_v[sl], mask=m)
            plsc.store_scatter(ca_v, [pos], ain_v[sl], mask=m)
            plsc.store_scatter(cl_v, [pos], sv - lo, mask=m)
            return cnt + plsc.all_reduce_population_count(m)[0]

        cnt = lax.fori_loop(0, SCAN // 16, grp, cnt, unroll=False)

        nb = cnt // 16

        @pl.when(nb > 0)
        def _():
            start_gather(0, rows_a, sem_ga)

        def pair(p, carry):
            b0 = 2 * p
            b1 = b0 + 1
            wait_gather(b0 * 16, rows_a, sem_ga)

            @pl.when(b1 < nb)
            def _():
                start_gather(b1 * 16, rows_b, sem_gb)

            accum_rows(b0 * 16, rows_a)

            @pl.when(b1 < nb)
            def _():
                wait_gather(b1 * 16, rows_b, sem_gb)

                @pl.when(b1 + 1 < nb)
                def _():
                    start_gather((b1 + 1) * 16, rows_a, sem_ga)

                accum_rows(b1 * 16, rows_b)

            return carry

        lax.fori_loop(0, (nb + 1) // 2, pair, 0, unroll=False)

        rem = cnt - nb * 16
        mv = lanes < rem
        src = pl.ds(nb * 16, 16)
        cg_v[pl.ds(0, 16)] = jnp.where(mv, cg_v[src], 0)
        ca_v[pl.ds(0, 16)] = jnp.where(mv, ca_v[src], 0.0)
        cl_v[pl.ds(0, 16)] = jnp.where(mv, cl_v[src], RPT)
        return rem

    bufs_a = (sin_a, gin_a, ain_a)
    bufs_b = (sin_b, gin_b, ain_b)

    start_win(0, bufs_a, sem_wa)

    def win_pair(p, cnt):
        w0 = 2 * p
        w1 = w0 + 1
        cnt = do_window(w0, cnt, bufs_a, sem_wa, w1, bufs_b, sem_wb)
        cnt = do_window(w1, cnt, bufs_b, sem_wb, w1 + 1, bufs_a, sem_wa)
        return cnt

    cnt = lax.fori_loop(0, NW // 2, win_pair, 0, unroll=False)

    @pl.when(cnt > 0)
    def _():
        pltpu.sync_copy(tbl_hbm.at[cg_v.at[pl.ds(0, 16)]], rows_a)
        accum_rows(0, rows_a)

    pltpu.sync_copy(acc_v.at[pl.ds(0, RPT)], out_hbm.at[c, pl.ds(lo, RPT)])


@jax.jit
def _sc_edge_call(tbl, gidx, sidx, alpha, zeros):
    mesh = plsc.VectorSubcoreMesh(core_axis_name="c", subcore_axis_name="s")
    kern = functools.partial(
        pl.kernel,
        mesh=mesh,
        compiler_params=pltpu.CompilerParams(needs_layout_passes=False),
        out_type=jax.ShapeDtypeStruct((2, NPAD, W), jnp.float32),
        scratch_types=[
            pltpu.VMEM((ACC_ROWS, W), jnp.float32),
            pltpu.VMEM((SCAN,), jnp.int32),
            pltpu.VMEM((SCAN,), jnp.int32),
            pltpu.VMEM((SCAN,), jnp.float32),
            pltpu.VMEM((SCAN,), jnp.int32),
            pltpu.VMEM((SCAN,), jnp.int32),
            pltpu.VMEM((SCAN,), jnp.float32),
            pltpu.VMEM((CAP,), jnp.int32),
            pltpu.VMEM((CAP,), jnp.float32),
            pltpu.VMEM((CAP,), jnp.int32),
            pltpu.VMEM((16, W), jnp.float32),
            pltpu.VMEM((16, W), jnp.float32),
            pltpu.SemaphoreType.DMA,
            pltpu.SemaphoreType.DMA,
            pltpu.SemaphoreType.DMA,
            pltpu.SemaphoreType.DMA,
        ],
    )(_sc_edge_kernel)
    return kern(tbl, gidx, sidx, alpha, zeros)


def _tc_finish_kernel(feat_ref, ns_ref, accA_ref, accM_ref,
                      w1t_ref, w2t_ref, b_ref, out_ref):
    fA = feat_ref[...] + accA_ref[...]
    fM = ns_ref[...] * accM_ref[...]
    h = jnp.dot(fA, w1t_ref[...], preferred_element_type=jnp.float32)
    h = h + jnp.dot(fM, w2t_ref[...], preferred_element_type=jnp.float32)
    h = h + b_ref[...]
    h = jnp.where(h >= 0, h, 0.2 * h)
    nrm = jnp.maximum(jnp.sqrt(jnp.sum(h * h, axis=1, keepdims=True)), 1e-12)
    out_ref[...] = h / nrm


@jax.jit
def _tc_finish_call(feat, ns, accA, accM, w1t, w2t, b):
    R = 1000
    nblk = (2 * N) // R
    blk = lambda i: (i, 0)
    return pl.pallas_call(
        _tc_finish_kernel,
        grid=(nblk,),
        in_specs=[
            pl.BlockSpec((R, D), blk),
            pl.BlockSpec((R, D), blk),
            pl.BlockSpec((R, D), blk),
            pl.BlockSpec((R, D), blk),
            pl.BlockSpec((D, D), lambda i: (0, 0)),
            pl.BlockSpec((D, D), lambda i: (0, 0)),
            pl.BlockSpec((1, D), lambda i: (0, 0)),
        ],
        out_specs=pl.BlockSpec((R, D), blk),
        out_shape=jax.ShapeDtypeStruct((2 * N, D), jnp.float32),
    )(feat, ns, accA, accM, w1t, w2t, b)


def kernel(feat_user, feat_item, ui_src, ui_dst, norm_ui, norm_iu,
           norm_user, norm_item, W1_w, W1_b, W2_w, W2_b):
    fu_s = feat_user * norm_user
    fi_s = feat_item * norm_item
    tbl = jnp.concatenate([
        jnp.concatenate([feat_item, fi_s], axis=1),
        jnp.concatenate([feat_user, fu_s], axis=1),
    ], axis=0)
    gidx = jnp.concatenate([ui_dst, ui_src + N])
    sidx = jnp.concatenate([ui_src, ui_dst])
    alpha = jnp.concatenate([norm_iu[:, 0], norm_ui[:, 0]])
    zeros = jnp.zeros((ACC_ROWS, W), jnp.float32)

    acc = _sc_edge_call(tbl, gidx, sidx, alpha, zeros)
    accA = acc[:, :N, :D].reshape(2 * N, D)
    accM = acc[:, :N, D:].reshape(2 * N, D)

    feat = jnp.concatenate([feat_user, feat_item], axis=0)
    ns = jnp.concatenate([fu_s, fi_s], axis=0)
    h = _tc_finish_call(feat, ns, accA, accM, W1_w.T, W2_w.T,
                        (W1_b + W2_b).reshape(1, D))
    return h[:N], h[N:]


# speedup vs baseline: 3.6315x; 2.2954x over previous
import functools

import jax
import jax.numpy as jnp
from jax import lax
from jax.experimental import pallas as pl
from jax.experimental.pallas import tpu as pltpu
from jax.experimental.pallas import tpu_sc as plsc

N = 5000
E = 320000
D = 128
W = 2 * D
RPT = 320
NPAD = 16 * RPT
ACC_ROWS = RPT + 8
SCAN = 3200
NW = E // SCAN
CAP = SCAN + 16


def _sc_edge_kernel(tbl_hbm, gidx_hbm, sidx_hbm, alpha_hbm, zeros_hbm,
                    out_hbm, acc_v, sin_a, gin_a, ain_a, sin_b, gin_b, ain_b,
                    cg_v, ca_v, cl_v, rows_a, rows_b,
                    sem_wa, sem_wb, sem_ga, sem_gb):
    c = lax.axis_index("c")
    s = lax.axis_index("s")
    lo = s * RPT
    lanes = lax.iota(jnp.int32, 16)

    pltpu.sync_copy(zeros_hbm, acc_v)

    def win_off(w):
        return pl.multiple_of(c * E + w * SCAN, SCAN)

    def start_win(w, bufs, sem):
        sv, gv, av = bufs
        off = win_off(w)
        pltpu.async_copy(sidx_hbm.at[pl.ds(off, SCAN)], sv, sem)
        pltpu.async_copy(gidx_hbm.at[pl.ds(off, SCAN)], gv, sem)
        pltpu.async_copy(alpha_hbm.at[pl.ds(off, SCAN)], av, sem)

    def wait_win(w, bufs, sem):
        sv, gv, av = bufs
        off = win_off(w)
        pltpu.make_async_copy(sidx_hbm.at[pl.ds(off, SCAN)], sv, sem).wait()
        pltpu.make_async_copy(gidx_hbm.at[pl.ds(off, SCAN)], gv, sem).wait()
        pltpu.make_async_copy(alpha_hbm.at[pl.ds(off, SCAN)], av, sem).wait()

    def start_gather(bo, rows, sem):
        pltpu.async_copy(tbl_hbm.at[cg_v.at[pl.ds(bo, 16)]], rows, sem)

    def wait_gather(bo, rows, sem):
        pltpu.make_async_copy(tbl_hbm.at[cg_v.at[pl.ds(bo, 16)]], rows,
                              sem).wait()

    def accum_rows(bo, rows):
        av = ca_v[pl.ds(bo, 16)]
        lsv = cl_v[pl.ds(bo, 16)]
        for l in range(16):
            a = av[l]
            ls = lsv[l]
            xs = []
            for j in range(16):
                x = rows[l, pl.ds(j * 16, 16)]
                if j < 8:
                    x = x * a
                xs.append(x)
            for j in range(16):
                plsc.addupdate(acc_v.at[ls, pl.ds(j * 16, 16)], xs[j])

    def do_window(w, cnt, cur_bufs, cur_sem, nxt_w, nxt_bufs, nxt_sem):
        sin_v, gin_v, ain_v = cur_bufs
        wait_win(w, cur_bufs, cur_sem)

        @pl.when(nxt_w < NW)
        def _():
            start_win(nxt_w, nxt_bufs, nxt_sem)

        def grp(g, cntv):
            datas = []
            for u in range(4):
                sl = pl.ds(g * 64 + u * 16, 16)
                sv = sin_v[sl]
                ls = sv - lo
                m = ls.astype(jnp.uint32) < jnp.uint32(RPT)
                pc = plsc.cumsum(jnp.where(m, 1, 0))
                datas.append((sl, ls, m, pc))
            for sl, ls, m, pc in datas:
                pos = cntv + pc - 1
                plsc.store_scatter(cg_v, [pos], gin_v[sl], mask=m)
                plsc.store_scatter(ca_v, [pos], ain_v[sl], mask=m)
                plsc.store_scatter(cl_v, [pos], ls, mask=m)
                cntv = cntv + pc[15]
            return cntv

        cntv = lax.fori_loop(0, SCAN // 64, grp,
                             jnp.broadcast_to(cnt, (16,)), unroll=False)
        cnt = cntv[0]

        nb = cnt // 16

        @pl.when(nb > 0)
        def _():
            start_gather(0, rows_a, sem_ga)

        def pair(p, carry):
            b0 = 2 * p
            b1 = b0 + 1
            wait_gather(b0 * 16, rows_a, sem_ga)

            @pl.when(b1 < nb)
            def _():
                start_gather(b1 * 16, rows_b, sem_gb)

            accum_rows(b0 * 16, rows_a)

            @pl.when(b1 < nb)
            def _():
                wait_gather(b1 * 16, rows_b, sem_gb)

                @pl.when(b1 + 1 < nb)
                def _():
                    start_gather((b1 + 1) * 16, rows_a, sem_ga)

                accum_rows(b1 * 16, rows_b)

            return carry

        lax.fori_loop(0, (nb + 1) // 2, pair, 0, unroll=False)

        rem = cnt - nb * 16
        mv = lanes < rem
        src = pl.ds(nb * 16, 16)
        cg_v[pl.ds(0, 16)] = jnp.where(mv, cg_v[src], 0)
        ca_v[pl.ds(0, 16)] = jnp.where(mv, ca_v[src], 0.0)
        cl_v[pl.ds(0, 16)] = jnp.where(mv, cl_v[src], RPT)
        return rem

    bufs_a = (sin_a, gin_a, ain_a)
    bufs_b = (sin_b, gin_b, ain_b)

    start_win(0, bufs_a, sem_wa)

    def win_pair(p, cnt):
        w0 = 2 * p
        w1 = w0 + 1
        cnt = do_window(w0, cnt, bufs_a, sem_wa, w1, bufs_b, sem_wb)
        cnt = do_window(w1, cnt, bufs_b, sem_wb, w1 + 1, bufs_a, sem_wa)
        return cnt

    cnt = lax.fori_loop(0, NW // 2, win_pair, 0, unroll=False)

    @pl.when(cnt > 0)
    def _():
        pltpu.sync_copy(tbl_hbm.at[cg_v.at[pl.ds(0, 16)]], rows_a)
        accum_rows(0, rows_a)

    pltpu.sync_copy(acc_v.at[pl.ds(0, RPT)], out_hbm.at[c, pl.ds(lo, RPT)])


@jax.jit
def _sc_edge_call(tbl, gidx, sidx, alpha, zeros):
    mesh = plsc.VectorSubcoreMesh(core_axis_name="c", subcore_axis_name="s")
    kern = functools.partial(
        pl.kernel,
        mesh=mesh,
        compiler_params=pltpu.CompilerParams(needs_layout_passes=False),
        out_type=jax.ShapeDtypeStruct((2, NPAD, W), jnp.float32),
        scratch_types=[
            pltpu.VMEM((ACC_ROWS, W), jnp.float32),
            pltpu.VMEM((SCAN,), jnp.int32),
            pltpu.VMEM((SCAN,), jnp.int32),
            pltpu.VMEM((SCAN,), jnp.float32),
            pltpu.VMEM((SCAN,), jnp.int32),
            pltpu.VMEM((SCAN,), jnp.int32),
            pltpu.VMEM((SCAN,), jnp.float32),
            pltpu.VMEM((CAP,), jnp.int32),
            pltpu.VMEM((CAP,), jnp.float32),
            pltpu.VMEM((CAP,), jnp.int32),
            pltpu.VMEM((16, W), jnp.float32),
            pltpu.VMEM((16, W), jnp.float32),
            pltpu.SemaphoreType.DMA,
            pltpu.SemaphoreType.DMA,
            pltpu.SemaphoreType.DMA,
            pltpu.SemaphoreType.DMA,
        ],
    )(_sc_edge_kernel)
    return kern(tbl, gidx, sidx, alpha, zeros)


def _tc_finish_kernel(feat_ref, ns_ref, accA_ref, accM_ref,
                      w1t_ref, w2t_ref, b_ref, out_ref):
    fA = feat_ref[...] + accA_ref[...]
    fM = ns_ref[...] * accM_ref[...]
    h = jnp.dot(fA, w1t_ref[...], preferred_element_type=jnp.float32)
    h = h + jnp.dot(fM, w2t_ref[...], preferred_element_type=jnp.float32)
    h = h + b_ref[...]
    h = jnp.where(h >= 0, h, 0.2 * h)
    nrm = jnp.maximum(jnp.sqrt(jnp.sum(h * h, axis=1, keepdims=True)), 1e-12)
    out_ref[...] = h / nrm


@jax.jit
def _tc_finish_call(feat, ns, accA, accM, w1t, w2t, b):
    R = 1000
    nblk = (2 * N) // R
    blk = lambda i: (i, 0)
    return pl.pallas_call(
        _tc_finish_kernel,
        grid=(nblk,),
        in_specs=[
            pl.BlockSpec((R, D), blk),
            pl.BlockSpec((R, D), blk),
            pl.BlockSpec((R, D), blk),
            pl.BlockSpec((R, D), blk),
            pl.BlockSpec((D, D), lambda i: (0, 0)),
            pl.BlockSpec((D, D), lambda i: (0, 0)),
            pl.BlockSpec((1, D), lambda i: (0, 0)),
        ],
        out_specs=pl.BlockSpec((R, D), blk),
        out_shape=jax.ShapeDtypeStruct((2 * N, D), jnp.float32),
    )(feat, ns, accA, accM, w1t, w2t, b)


def kernel(feat_user, feat_item, ui_src, ui_dst, norm_ui, norm_iu,
           norm_user, norm_item, W1_w, W1_b, W2_w, W2_b):
    fu_s = feat_user * norm_user
    fi_s = feat_item * norm_item
    tbl = jnp.concatenate([
        jnp.concatenate([feat_item, fi_s], axis=1),
        jnp.concatenate([feat_user, fu_s], axis=1),
    ], axis=0)
    gidx = jnp.concatenate([ui_dst, ui_src + N])
    sidx = jnp.concatenate([ui_src, ui_dst])
    alpha = jnp.concatenate([norm_iu[:, 0], norm_ui[:, 0]])
    zeros = jnp.zeros((ACC_ROWS, W), jnp.float32)

    acc = _sc_edge_call(tbl, gidx, sidx, alpha, zeros)
    accA = acc[:, :N, :D].reshape(2 * N, D)
    accM = acc[:, :N, D:].reshape(2 * N, D)

    feat = jnp.concatenate([feat_user, feat_item], axis=0)
    ns = jnp.concatenate([fu_s, fi_s], axis=0)
    h = _tc_finish_call(feat, ns, accA, accM, W1_w.T, W2_w.T,
                        (W1_b + W2_b).reshape(1, D))
    return h[:N], h[N:]


# speedup vs baseline: 3.6344x; 1.0008x over previous
import functools

import jax
import jax.numpy as jnp
from jax import lax
from jax.experimental import pallas as pl
from jax.experimental.pallas import tpu as pltpu
from jax.experimental.pallas import tpu_sc as plsc

N = 5000
E = 320000
D = 128
W = 2 * D
RPT = 320
NPAD = 16 * RPT
ACC_ROWS = RPT + 8
SCAN = 3200
NW = E // SCAN
CAP = SCAN + 16


def _sc_edge_kernel(tbl_hbm, gidx_hbm, sidx_hbm, alpha_hbm, zeros_hbm,
                    out_hbm, acc_v, sin_a, gin_a, ain_a, sin_b, gin_b, ain_b,
                    cg_v, ca_v, cl_v, rows_a, rows_b,
                    sem_wa, sem_wb, sem_ga, sem_gb):
    c = lax.axis_index("c")
    s = lax.axis_index("s")
    lo = s * RPT
    lanes = lax.iota(jnp.int32, 16)

    pltpu.sync_copy(zeros_hbm, acc_v)

    def win_off(w):
        return pl.multiple_of(c * E + w * SCAN, SCAN)

    def start_win(w, bufs, sem):
        sv, gv, av = bufs
        off = win_off(w)
        pltpu.async_copy(sidx_hbm.at[pl.ds(off, SCAN)], sv, sem)
        pltpu.async_copy(gidx_hbm.at[pl.ds(off, SCAN)], gv, sem)
        pltpu.async_copy(alpha_hbm.at[pl.ds(off, SCAN)], av, sem)

    def wait_win(w, bufs, sem):
        sv, gv, av = bufs
        off = win_off(w)
        pltpu.make_async_copy(sidx_hbm.at[pl.ds(off, SCAN)], sv, sem).wait()
        pltpu.make_async_copy(gidx_hbm.at[pl.ds(off, SCAN)], gv, sem).wait()
        pltpu.make_async_copy(alpha_hbm.at[pl.ds(off, SCAN)], av, sem).wait()

    def start_gather(bo, rows, sem):
        pltpu.async_copy(tbl_hbm.at[cg_v.at[pl.ds(bo, 16)]], rows, sem)

    def wait_gather(bo, rows, sem):
        pltpu.make_async_copy(tbl_hbm.at[cg_v.at[pl.ds(bo, 16)]], rows,
                              sem).wait()

    def accum_rows(bo, rows):
        av = ca_v[pl.ds(bo, 16)]
        lsv = cl_v[pl.ds(bo, 16)]

        def load_edge(l):
            a = av[l]
            xs = []
            for j in range(16):
                x = rows[l, pl.ds(j * 16, 16)]
                if j < 8:
                    x = x * a
                xs.append(x)
            return xs

        def store_edge(l, xs):
            ls = lsv[l]
            for j in range(16):
                plsc.addupdate(acc_v.at[ls, pl.ds(j * 16, 16)], xs[j])

        prev = load_edge(0)
        for l in range(1, 16):
            cur = load_edge(l)
            store_edge(l - 1, prev)
            prev = cur
        store_edge(15, prev)

    def do_window(w, cnt, cur_bufs, cur_sem, nxt_w, nxt_bufs, nxt_sem):
        sin_v, gin_v, ain_v = cur_bufs
        wait_win(w, cur_bufs, cur_sem)

        @pl.when(nxt_w < NW)
        def _():
            start_win(nxt_w, nxt_bufs, nxt_sem)

        def grp(g, cntv):
            datas = []
            for u in range(4):
                sl = pl.ds(g * 64 + u * 16, 16)
                sv = sin_v[sl]
                ls = sv - lo
                m = ls.astype(jnp.uint32) < jnp.uint32(RPT)
                pc = plsc.cumsum(jnp.where(m, 1, 0))
                datas.append((sl, ls, m, pc))
            for sl, ls, m, pc in datas:
                pos = cntv + pc - 1
                plsc.store_scatter(cg_v, [pos], gin_v[sl], mask=m)
                plsc.store_scatter(ca_v, [pos], ain_v[sl], mask=m)
                plsc.store_scatter(cl_v, [pos], ls, mask=m)
                cntv = cntv + pc[15]
            return cntv

        cntv = lax.fori_loop(0, SCAN // 64, grp,
                             jnp.broadcast_to(cnt, (16,)), unroll=False)
        cnt = cntv[0]

        nb = cnt // 16

        @pl.when(nb > 0)
        def _():
            start_gather(0, rows_a, sem_ga)

        def pair(p, carry):
            b0 = 2 * p
            b1 = b0 + 1
            wait_gather(b0 * 16, rows_a, sem_ga)

            @pl.when(b1 < nb)
            def _():
                start_gather(b1 * 16, rows_b, sem_gb)

            accum_rows(b0 * 16, rows_a)

            @pl.when(b1 < nb)
            def _():
                wait_gather(b1 * 16, rows_b, sem_gb)

                @pl.when(b1 + 1 < nb)
                def _():
                    start_gather((b1 + 1) * 16, rows_a, sem_ga)

                accum_rows(b1 * 16, rows_b)

            return carry

        lax.fori_loop(0, (nb + 1) // 2, pair, 0, unroll=False)

        rem = cnt - nb * 16
        mv = lanes < rem
        src = pl.ds(nb * 16, 16)
        cg_v[pl.ds(0, 16)] = jnp.where(mv, cg_v[src], 0)
        ca_v[pl.ds(0, 16)] = jnp.where(mv, ca_v[src], 0.0)
        cl_v[pl.ds(0, 16)] = jnp.where(mv, cl_v[src], RPT)
        return rem

    bufs_a = (sin_a, gin_a, ain_a)
    bufs_b = (sin_b, gin_b, ain_b)

    start_win(0, bufs_a, sem_wa)

    def win_pair(p, cnt):
        w0 = 2 * p
        w1 = w0 + 1
        cnt = do_window(w0, cnt, bufs_a, sem_wa, w1, bufs_b, sem_wb)
        cnt = do_window(w1, cnt, bufs_b, sem_wb, w1 + 1, bufs_a, sem_wa)
        return cnt

    cnt = lax.fori_loop(0, NW // 2, win_pair, 0, unroll=False)

    @pl.when(cnt > 0)
    def _():
        pltpu.sync_copy(tbl_hbm.at[cg_v.at[pl.ds(0, 16)]], rows_a)
        accum_rows(0, rows_a)

    pltpu.sync_copy(acc_v.at[pl.ds(0, RPT)], out_hbm.at[c, pl.ds(lo, RPT)])


@jax.jit
def _sc_edge_call(tbl, gidx, sidx, alpha, zeros):
    mesh = plsc.VectorSubcoreMesh(core_axis_name="c", subcore_axis_name="s")
    kern = functools.partial(
        pl.kernel,
        mesh=mesh,
        compiler_params=pltpu.CompilerParams(needs_layout_passes=False),
        out_type=jax.ShapeDtypeStruct((2, NPAD, W), jnp.float32),
        scratch_types=[
            pltpu.VMEM((ACC_ROWS, W), jnp.float32),
            pltpu.VMEM((SCAN,), jnp.int32),
            pltpu.VMEM((SCAN,), jnp.int32),
            pltpu.VMEM((SCAN,), jnp.float32),
            pltpu.VMEM((SCAN,), jnp.int32),
            pltpu.VMEM((SCAN,), jnp.int32),
            pltpu.VMEM((SCAN,), jnp.float32),
            pltpu.VMEM((CAP,), jnp.int32),
            pltpu.VMEM((CAP,), jnp.float32),
            pltpu.VMEM((CAP,), jnp.int32),
            pltpu.VMEM((16, W), jnp.float32),
            pltpu.VMEM((16, W), jnp.float32),
            pltpu.SemaphoreType.DMA,
            pltpu.SemaphoreType.DMA,
            pltpu.SemaphoreType.DMA,
            pltpu.SemaphoreType.DMA,
        ],
    )(_sc_edge_kernel)
    return kern(tbl, gidx, sidx, alpha, zeros)


def _tc_finish_kernel(feat_ref, ns_ref, accA_ref, accM_ref,
                      w1t_ref, w2t_ref, b_ref, out_ref):
    fA = feat_ref[...] + accA_ref[...]
    fM = ns_ref[...] * accM_ref[...]
    h = jnp.dot(fA, w1t_ref[...], preferred_element_type=jnp.float32)
    h = h + jnp.dot(fM, w2t_ref[...], preferred_element_type=jnp.float32)
    h = h + b_ref[...]
    h = jnp.where(h >= 0, h, 0.2 * h)
    nrm = jnp.maximum(jnp.sqrt(jnp.sum(h * h, axis=1, keepdims=True)), 1e-12)
    out_ref[...] = h / nrm


@jax.jit
def _tc_finish_call(feat, ns, accA, accM, w1t, w2t, b):
    R = 1000
    nblk = (2 * N) // R
    blk = lambda i: (i, 0)
    return pl.pallas_call(
        _tc_finish_kernel,
        grid=(nblk,),
        in_specs=[
            pl.BlockSpec((R, D), blk),
            pl.BlockSpec((R, D), blk),
            pl.BlockSpec((R, D), blk),
            pl.BlockSpec((R, D), blk),
            pl.BlockSpec((D, D), lambda i: (0, 0)),
            pl.BlockSpec((D, D), lambda i: (0, 0)),
            pl.BlockSpec((1, D), lambda i: (0, 0)),
        ],
        out_specs=pl.BlockSpec((R, D), blk),
        out_shape=jax.ShapeDtypeStruct((2 * N, D), jnp.float32),
    )(feat, ns, accA, accM, w1t, w2t, b)


def kernel(feat_user, feat_item, ui_src, ui_dst, norm_ui, norm_iu,
           norm_user, norm_item, W1_w, W1_b, W2_w, W2_b):
    fu_s = feat_user * norm_user
    fi_s = feat_item * norm_item
    tbl = jnp.concatenate([
        jnp.concatenate([feat_item, fi_s], axis=1),
        jnp.concatenate([feat_user, fu_s], axis=1),
    ], axis=0)
    gidx = jnp.concatenate([ui_dst, ui_src + N])
    sidx = jnp.concatenate([ui_src, ui_dst])
    alpha = jnp.concatenate([norm_iu[:, 0], norm_ui[:, 0]])
    zeros = jnp.zeros((ACC_ROWS, W), jnp.float32)

    acc = _sc_edge_call(tbl, gidx, sidx, alpha, zeros)
    accA = acc[:, :N, :D].reshape(2 * N, D)
    accM = acc[:, :N, D:].reshape(2 * N, D)

    feat = jnp.concatenate([feat_user, feat_item], axis=0)
    ns = jnp.concatenate([fu_s, fi_s], axis=0)
    h = _tc_finish_call(feat, ns, accA, accM, W1_w.T, W2_w.T,
                        (W1_b + W2_b).reshape(1, D))
    return h[:N], h[N:]


# speedup vs baseline: 4.0639x; 1.1182x over previous
import functools

import jax
import jax.numpy as jnp
from jax import lax
from jax.experimental import pallas as pl
from jax.experimental.pallas import tpu as pltpu
from jax.experimental.pallas import tpu_sc as plsc

N = 5000
E = 320000
D = 128
W = 2 * D
RPT = 320
NPAD = 16 * RPT
ACC_ROWS = RPT + 8
SCAN = 1600
NW = E // SCAN
CAP = SCAN + 16


def _sc_edge_kernel(tbl_hbm, gidx_hbm, sidx_hbm, alpha_hbm, beta_hbm,
                    zeros_hbm, out_hbm, acc_v, beta_v,
                    sin_a, gin_a, ain_a, sin_b, gin_b, ain_b,
                    cg_v, ca_v, cl_v, rows_a, rows_b,
                    sem_wa, sem_wb, sem_ga, sem_gb):
    c = lax.axis_index("c")
    s = lax.axis_index("s")
    lo = s * RPT
    lanes = lax.iota(jnp.int32, 16)

    boff = pl.multiple_of(c * NPAD, NPAD)
    pltpu.sync_copy(beta_hbm.at[pl.ds(boff, NPAD)], beta_v)
    pltpu.sync_copy(zeros_hbm, acc_v)

    def win_off(w):
        return pl.multiple_of(c * E + w * SCAN, SCAN)

    def start_win(w, bufs, sem):
        sv, gv, av = bufs
        off = win_off(w)
        pltpu.async_copy(sidx_hbm.at[pl.ds(off, SCAN)], sv, sem)
        pltpu.async_copy(gidx_hbm.at[pl.ds(off, SCAN)], gv, sem)
        pltpu.async_copy(alpha_hbm.at[pl.ds(off, SCAN)], av, sem)

    def wait_win(w, bufs, sem):
        sv, gv, av = bufs
        off = win_off(w)
        pltpu.make_async_copy(sidx_hbm.at[pl.ds(off, SCAN)], sv, sem).wait()
        pltpu.make_async_copy(gidx_hbm.at[pl.ds(off, SCAN)], gv, sem).wait()
        pltpu.make_async_copy(alpha_hbm.at[pl.ds(off, SCAN)], av, sem).wait()

    def start_gather(bo, rows, sem):
        pltpu.async_copy(tbl_hbm.at[c].at[cg_v.at[pl.ds(bo, 16)]], rows, sem)

    def wait_gather(bo, rows, sem):
        pltpu.make_async_copy(tbl_hbm.at[c].at[cg_v.at[pl.ds(bo, 16)]], rows,
                              sem).wait()

    def accum_rows(bo, rows):
        av = ca_v[pl.ds(bo, 16)]
        gv = cg_v[pl.ds(bo, 16)]
        lsv = cl_v[pl.ds(bo, 16)]
        bv = plsc.load_gather(beta_v, [gv])

        def load_edge(l):
            a = av[l]
            b = bv[l]
            xs = []
            for j in range(8):
                x = rows[l, pl.ds(j * 16, 16)]
                xs.append(x * a)
            for j in range(8):
                x = rows[l, pl.ds(j * 16, 16)]
                xs.append(x * b)
            return xs

        def store_edge(l, xs):
            ls = lsv[l]
            for j in range(16):
                plsc.addupdate(acc_v.at[ls, pl.ds(j * 16, 16)], xs[j])

        prev = load_edge(0)
        for l in range(1, 16):
            cur = load_edge(l)
            store_edge(l - 1, prev)
            prev = cur
        store_edge(15, prev)

    def do_window(w, cnt, cur_bufs, cur_sem, nxt_w, nxt_bufs, nxt_sem):
        sin_v, gin_v, ain_v = cur_bufs
        wait_win(w, cur_bufs, cur_sem)

        @pl.when(nxt_w < NW)
        def _():
            start_win(nxt_w, nxt_bufs, nxt_sem)

        def grp(g, cntv):
            datas = []
            for u in range(4):
                sl = pl.ds(g * 64 + u * 16, 16)
                sv = sin_v[sl]
                ls = sv - lo
                m = ls.astype(jnp.uint32) < jnp.uint32(RPT)
                pc = plsc.cumsum(jnp.where(m, 1, 0))
                datas.append((sl, ls, m, pc))
            for sl, ls, m, pc in datas:
                pos = cntv + pc - 1
                plsc.store_scatter(cg_v, [pos], gin_v[sl], mask=m)
                plsc.store_scatter(ca_v, [pos], ain_v[sl], mask=m)
                plsc.store_scatter(cl_v, [pos], ls, mask=m)
                cntv = cntv + pc[15]
            return cntv

        cntv = lax.fori_loop(0, SCAN // 64, grp,
                             jnp.broadcast_to(cnt, (16,)), unroll=False)
        cnt = cntv[0]

        nb = cnt // 16

        @pl.when(nb > 0)
        def _():
            start_gather(0, rows_a, sem_ga)

        def pair(p, carry):
            b0 = 2 * p
            b1 = b0 + 1
            wait_gather(b0 * 16, rows_a, sem_ga)

            @pl.when(b1 < nb)
            def _():
                start_gather(b1 * 16, rows_b, sem_gb)

            accum_rows(b0 * 16, rows_a)

            @pl.when(b1 < nb)
            def _():
                wait_gather(b1 * 16, rows_b, sem_gb)

                @pl.when(b1 + 1 < nb)
                def _():
                    start_gather((b1 + 1) * 16, rows_a, sem_ga)

                accum_rows(b1 * 16, rows_b)

            return carry

        lax.fori_loop(0, (nb + 1) // 2, pair, 0, unroll=False)

        rem = cnt - nb * 16
        mv = lanes < rem
        src = pl.ds(nb * 16, 16)
        cg_v[pl.ds(0, 16)] = jnp.where(mv, cg_v[src], 0)
        ca_v[pl.ds(0, 16)] = jnp.where(mv, ca_v[src], 0.0)
        cl_v[pl.ds(0, 16)] = jnp.where(mv, cl_v[src], RPT)
        return rem

    bufs_a = (sin_a, gin_a, ain_a)
    bufs_b = (sin_b, gin_b, ain_b)

    start_win(0, bufs_a, sem_wa)

    def win_pair(p, cnt):
        w0 = 2 * p
        w1 = w0 + 1
        cnt = do_window(w0, cnt, bufs_a, sem_wa, w1, bufs_b, sem_wb)
        cnt = do_window(w1, cnt, bufs_b, sem_wb, w1 + 1, bufs_a, sem_wa)
        return cnt

    cnt = lax.fori_loop(0, NW // 2, win_pair, 0, unroll=False)

    @pl.when(cnt > 0)
    def _():
        pltpu.sync_copy(tbl_hbm.at[c].at[cg_v.at[pl.ds(0, 16)]], rows_a)
        accum_rows(0, rows_a)

    pltpu.sync_copy(acc_v.at[pl.ds(0, RPT)], out_hbm.at[c, pl.ds(lo, RPT)])


@jax.jit
def _sc_edge_call(tbl, gidx, sidx, alpha, beta, zeros):
    mesh = plsc.VectorSubcoreMesh(core_axis_name="c", subcore_axis_name="s")
    kern = functools.partial(
        pl.kernel,
        mesh=mesh,
        compiler_params=pltpu.CompilerParams(needs_layout_passes=False),
        out_type=jax.ShapeDtypeStruct((2, NPAD, W), jnp.float32),
        scratch_types=[
            pltpu.VMEM((ACC_ROWS, W), jnp.float32),
            pltpu.VMEM((NPAD,), jnp.float32),
            pltpu.VMEM((SCAN,), jnp.int32),
            pltpu.VMEM((SCAN,), jnp.int32),
            pltpu.VMEM((SCAN,), jnp.float32),
            pltpu.VMEM((SCAN,), jnp.int32),
            pltpu.VMEM((SCAN,), jnp.int32),
            pltpu.VMEM((SCAN,), jnp.float32),
            pltpu.VMEM((CAP,), jnp.int32),
            pltpu.VMEM((CAP,), jnp.float32),
            pltpu.VMEM((CAP,), jnp.int32),
            pltpu.VMEM((16, D), jnp.float32),
            pltpu.VMEM((16, D), jnp.float32),
            pltpu.SemaphoreType.DMA,
            pltpu.SemaphoreType.DMA,
            pltpu.SemaphoreType.DMA,
            pltpu.SemaphoreType.DMA,
        ],
    )(_sc_edge_kernel)
    return kern(tbl, gidx, sidx, alpha, beta, zeros)


def _tc_finish_kernel(feat_ref, ns_ref, accA_ref, accM_ref,
                      w1t_ref, w2t_ref, b_ref, out_ref):
    fA = feat_ref[...] + accA_ref[...]
    fM = ns_ref[...] * accM_ref[...]
    h = jnp.dot(fA, w1t_ref[...], preferred_element_type=jnp.float32)
    h = h + jnp.dot(fM, w2t_ref[...], preferred_element_type=jnp.float32)
    h = h + b_ref[...]
    h = jnp.where(h >= 0, h, 0.2 * h)
    nrm = jnp.maximum(jnp.sqrt(jnp.sum(h * h, axis=1, keepdims=True)), 1e-12)
    out_ref[...] = h / nrm


@jax.jit
def _tc_finish_call(feat, ns, accA, accM, w1t, w2t, b):
    R = 1000
    nblk = (2 * N) // R
    blk = lambda i: (i, 0)
    return pl.pallas_call(
        _tc_finish_kernel,
        grid=(nblk,),
        in_specs=[
            pl.BlockSpec((R, D), blk),
            pl.BlockSpec((R, D), blk),
            pl.BlockSpec((R, D), blk),
            pl.BlockSpec((R, D), blk),
            pl.BlockSpec((D, D), lambda i: (0, 0)),
            pl.BlockSpec((D, D), lambda i: (0, 0)),
            pl.BlockSpec((1, D), lambda i: (0, 0)),
        ],
        out_specs=pl.BlockSpec((R, D), blk),
        out_shape=jax.ShapeDtypeStruct((2 * N, D), jnp.float32),
    )(feat, ns, accA, accM, w1t, w2t, b)


def kernel(feat_user, feat_item, ui_src, ui_dst, norm_ui, norm_iu,
           norm_user, norm_item, W1_w, W1_b, W2_w, W2_b):
    pad = jnp.zeros((NPAD - N, D), jnp.float32)
    padv = jnp.zeros((NPAD - N,), jnp.float32)
    tbl = jnp.stack([
        jnp.concatenate([feat_item, pad], axis=0),
        jnp.concatenate([feat_user, pad], axis=0),
    ])
    beta = jnp.concatenate([norm_item[:, 0], padv, norm_user[:, 0], padv])
    gidx = jnp.concatenate([ui_dst, ui_src])
    sidx = jnp.concatenate([ui_src, ui_dst])
    alpha = jnp.concatenate([norm_iu[:, 0], norm_ui[:, 0]])
    zeros = jnp.zeros((ACC_ROWS, W), jnp.float32)

    acc = _sc_edge_call(tbl, gidx, sidx, alpha, beta, zeros)
    accA = acc[:, :N, :D].reshape(2 * N, D)
    accM = acc[:, :N, D:].reshape(2 * N, D)

    feat = jnp.concatenate([feat_user, feat_item], axis=0)
    ns = jnp.concatenate([feat_user * norm_user, feat_item * norm_item],
                         axis=0)
    h = _tc_finish_call(feat, ns, accA, accM, W1_w.T, W2_w.T,
                        (W1_b + W2_b).reshape(1, D))
    return h[:N], h[N:]
